# baseline (speedup 1.0000x reference)
import functools

import jax
import jax.numpy as jnp
from jax import lax
from jax.experimental import pallas as pl
from jax.experimental.pallas import tpu as pltpu
from jax.experimental.pallas import tpu_sc as plsc

NC = 2
NS = 16
L = 16
NW = NC * NS

BATCH = 16384
H_FEAT = 64
N_CLASSES = 2
BPW = BATCH // NW
NNODE = 1_000_000
NTC_FULL = 7812
TAIL0 = NTC_FULL * 128
NSLAB = 245
CAP = 1280
VROWS = CAP // 2
FLUSH = 64

_mesh = plsc.VectorSubcoreMesh(core_axis_name="c", subcore_axis_name="s")
_cparams = pltpu.CompilerParams(
    needs_layout_passes=False, use_tc_tiling_on_sc=True)


def _start_tc(w):
    return (w * NTC_FULL) // NW


def _phase_a_body(embT_hbm, tail_hbm, ids_hbm, out_hbm,
                  ids_v, tail_v, slab_v, colbuf, sem):
    wid = lax.axis_index("s") * NC + lax.axis_index("c")
    pltpu.sync_copy(ids_hbm.at[wid], ids_v)
    pltpu.sync_copy(tail_hbm, tail_v)

    lane = lax.iota(jnp.int32, L)
    stc = _start_tc(wid)

    def fire(c, slot):
        tc = stc + jnp.minimum(c, NSLAB - 1)
        pltpu.async_copy(embT_hbm.at[:, pl.ds(tc * 128, 128)],
                         slab_v.at[slot], sem)

    def drain(slot):
        pltpu.make_async_copy(embT_hbm.at[:, pl.ds(0, 128)],
                              slab_v.at[slot], sem).wait()

    def read_id(p):
        pa = (p // L) * L
        vec = ids_v[pl.ds(pa, L)]
        sel = jnp.where(lane == (p % L), vec, 0)
        return jnp.sum(sel), sel

    def extract(src_ref_kind, p, idv):
        pr = p % (FLUSH * 2)
        row = pr // 2
        off = (pr % 2) * H_FEAT
        if src_ref_kind == "tail":
            r = idv - TAIL0
            for k in range(H_FEAT // L):
                colbuf[row, pl.ds(off + k * L, L)] = (
                    tail_v[r, pl.ds(k * L, L)])
        else:
            pass

    def flush(pf, wr):
        pltpu.sync_copy(
            colbuf, out_hbm.at[pl.ds(wid * VROWS + wr, FLUSH)])
        del pf

    def slab_join(c, carry):
        p, pf = carry
        slot = c % 2
        fire(c + 1, (c + 1) % 2)
        drain(slot)
        slab_end = (stc + c + 1) * 128

        def cond(st):
            pp, _ = st
            idv, _ = read_id(pp)
            return idv < slab_end

        def body(st):
            pp, _ = st
            idv, _ = read_id(pp)
            l = idv - (stc + c) * 128
            pr = pp % (FLUSH * 4)
            row = pr // 2
            off = (pr % 2) * H_FEAT
            for k in range(H_FEAT // L):
                ch = plsc.load_gather(
                    slab_v, [jnp.full((L,), slot, jnp.int32),
                             k * L + lane,
                             jnp.full((L,), l, jnp.int32)])
                colbuf[row, pl.ds(off + k * L, L)] = ch
            return pp + 1, 0

        p, _ = lax.while_loop(cond, body, (p, 0))

        def fcond(st):
            pp, pfx = st
            return pfx + FLUSH * 2 <= pp

        def fbody(st):
            pp, pfx = st
            win = pl.multiple_of((pfx % (FLUSH * 4)) // 2, FLUSH)
            pltpu.sync_copy(
                colbuf.at[pl.ds(win, FLUSH)],
                out_hbm.at[pl.ds(
                    pl.multiple_of(wid * VROWS + pfx // 2, 8), FLUSH)])
            return pp, pfx + FLUSH * 2

        p, pf = lax.while_loop(fcond, fbody, (p, pf))
        return p, pf

    fire(0, 0)
    p, pf = lax.fori_loop(0, NSLAB, slab_join, (0, 0))
    drain((NSLAB) % 2)

    def tcond(st):
        pp, _ = st
        idv, _ = read_id(pp)
        return idv < (1 << 29)

    def tbody(st):
        pp, _ = st
        idv, _ = read_id(pp)
        r = idv - TAIL0
        pr = pp % (FLUSH * 4)
        row = pr // 2
        off = (pr % 2) * H_FEAT
        for k in range(H_FEAT // L):
            ch = plsc.load_gather(
                tail_v, [jnp.full((L,), r, jnp.int32), k * L + lane])
            colbuf[row, pl.ds(off + k * L, L)] = ch
        return pp + 1, 0

    p, _ = lax.while_loop(tcond, tbody, (p, 0))

    @pl.when(pf < p)
    def _():
        win = pl.multiple_of((pf % (FLUSH * 4)) // 2, FLUSH)
        pltpu.sync_copy(
            colbuf.at[pl.ds(win, FLUSH)],
            out_hbm.at[pl.ds(
                pl.multiple_of(wid * VROWS + pf // 2, 8), FLUSH)])

    @pl.when(pf + FLUSH * 2 < p)
    def _():
        pf2 = pf + FLUSH * 2
        win = pl.multiple_of((pf2 % (FLUSH * 4)) // 2, FLUSH)
        pltpu.sync_copy(
            colbuf.at[pl.ds(win, FLUSH)],
            out_hbm.at[pl.ds(
                pl.multiple_of(wid * VROWS + pf2 // 2, 8), FLUSH)])


_phase_a = functools.partial(
    pl.kernel,
    out_type=jax.ShapeDtypeStruct((NW * VROWS, 128), jnp.float32),
    mesh=_mesh,
    compiler_params=_cparams,
    scratch_types=[
        pltpu.VMEM((CAP,), jnp.int32),
        pltpu.VMEM((64, H_FEAT), jnp.float32),
        pltpu.VMEM((2, H_FEAT, 128), jnp.float32),
        pltpu.VMEM((2 * FLUSH, 128), jnp.float32),
        pltpu.SemaphoreType.DMA,
    ],
)(_phase_a_body)


def _phase_b_body(vals_hbm, hsrc_hbm, hdst_hbm, w_hbm, b_hbm, out_hbm,
                  hs_v, hd_v, idx_v, sgat, dgat, wv, bv, outv, sem):
    wid = lax.axis_index("s") * NC + lax.axis_index("c")
    base = wid * BPW
    pltpu.sync_copy(hsrc_hbm.at[pl.ds(base, BPW)], hs_v)
    pltpu.sync_copy(hdst_hbm.at[pl.ds(base, BPW)], hd_v)
    pltpu.sync_copy(w_hbm, wv)
    pltpu.sync_copy(b_hbm, bv)

    bvec = bv[...]
    nck = H_FEAT // L
    w0c = [wv[0, pl.ds(k * L, L)] for k in range(nck)]
    w1c = [wv[1, pl.ds(k * L, L)] for k in range(nck)]
    b0vec = jnp.full((L,), bvec[0], jnp.float32)
    b1vec = jnp.full((L,), bvec[1], jnp.float32)
    lane = lax.iota(jnp.int32, L)
    mask_hi = lane == (L - 1)

    def mkrows(i, carry):
        hs = hs_v[pl.ds(i * L, L)]
        hd = hd_v[pl.ds(i * L, L)]
        idx_v[0, pl.ds(i * L, L)] = hs // 2
        idx_v[1, pl.ds(i * L, L)] = hd // 2
        return carry

    lax.fori_loop(0, BPW // L, mkrows, 0)

    NCH = BPW // 128

    def chunk(c, carry):
        pltpu.async_copy(vals_hbm.at[idx_v.at[0, pl.ds(c * 128, 128)]],
                         sgat, sem)
        pltpu.async_copy(vals_hbm.at[idx_v.at[1, pl.ds(c * 128, 128)]],
                         dgat, sem)
        pltpu.make_async_copy(vals_hbm.at[pl.ds(0, 128)], sgat, sem).wait()
        pltpu.make_async_copy(vals_hbm.at[pl.ds(0, 128)], dgat, sem).wait()

        def grp(g, carry2):
            e0 = g * L
            hsv = hs_v[pl.ds(c * 128 + e0, L)]
            hdv = hd_v[pl.ds(c * 128 + e0, L)]
            for ln in range(L):
                soff = (hsv[ln] % 2) * H_FEAT
                doff = (hdv[ln] % 2) * H_FEAT
                t0 = None
                t1 = None
                for k in range(nck):
                    cd = (sgat[e0 + ln, pl.ds(soff + k * L, L)]
                          * dgat[e0 + ln, pl.ds(doff + k * L, L)])
                    p0 = cd * w0c[k]
                    p1 = cd * w1c[k]
                    t0 = p0 if t0 is None else t0 + p0
                    t1 = p1 if t1 is None else t1 + p1
                cs0 = plsc.cumsum(t0) + b0vec
                cs1 = plsc.cumsum(t1) + b1vec
                oidx = jnp.full(
                    (L,), N_CLASSES * (c * 128 + e0 + ln), jnp.int32)
                plsc.store_scatter(outv, [oidx], cs0, mask=mask_hi)
                plsc.store_scatter(outv, [oidx + 1], cs1, mask=mask_hi)
            return carry2

        lax.fori_loop(0, 128 // L, grp, 0)
        return carry

    lax.fori_loop(0, NCH, chunk, 0)
    pltpu.sync_copy(outv, out_hbm.at[pl.ds(wid * BPW * N_CLASSES,
                                           BPW * N_CLASSES)])


_phase_b = functools.partial(
    pl.kernel,
    out_type=jax.ShapeDtypeStruct((BATCH * N_CLASSES,), jnp.float32),
    mesh=_mesh,
    compiler_params=_cparams,
    scratch_types=[
        pltpu.VMEM((BPW,), jnp.int32),
        pltpu.VMEM((BPW,), jnp.int32),
        pltpu.VMEM((2, BPW), jnp.int32),
        pltpu.VMEM((128, 128), jnp.float32),
        pltpu.VMEM((128, 128), jnp.float32),
        pltpu.VMEM((N_CLASSES, H_FEAT), jnp.float32),
        pltpu.VMEM((L,), jnp.float32),
        pltpu.VMEM((BATCH * N_CLASSES // NW,), jnp.float32),
        pltpu.SemaphoreType.DMA,
    ],
)(_phase_b_body)


def kernel(src_id, dst_id, embedding, W, b):
    src_id = src_id.astype(jnp.int32)
    dst_id = dst_id.astype(jnp.int32)
    ids_all = jnp.concatenate([src_id, dst_id])
    order = jnp.argsort(ids_all)
    ids_s = ids_all[order]

    bounds = (jnp.arange(NW, dtype=jnp.int32) * NTC_FULL) // NW * 128
    starts = jnp.searchsorted(ids_s, bounds).astype(jnp.int32)

    rel = jnp.arange(CAP, dtype=jnp.int32)
    ends = jnp.concatenate(
        [starts[1:], jnp.array([2 * BATCH], jnp.int32)])
    gidx = starts[:, None] + rel[None, :]
    valid = gidx < ends[:, None]
    ids_pad = jnp.where(valid,
                        ids_s[jnp.clip(gidx, 0, 2 * BATCH - 1)],
                        jnp.int32(1 << 29))

    w_of_p = jnp.clip(
        jnp.searchsorted(bounds, ids_s, side="right") - 1, 0, NW - 1
    ).astype(jnp.int32)
    p_arange = jnp.arange(2 * BATCH, dtype=jnp.int32)
    h_of_p = p_arange - starts[w_of_p] + w_of_p * CAP
    h_orig = jnp.zeros((2 * BATCH,), jnp.int32).at[order].set(h_of_p)
    h_src = h_orig[:BATCH]
    h_dst = h_orig[BATCH:]

    tail_rows = lax.slice(embedding, (TAIL0, 0), (NNODE, H_FEAT))
    vals = _phase_a(embedding.T, tail_rows, ids_pad)
    b_pad = jnp.zeros((L,), jnp.float32).at[:N_CLASSES].set(b)
    out_flat = _phase_b(vals, h_src, h_dst, W, b_pad)
    return out_flat.reshape(BATCH, N_CLASSES)


# speedup vs baseline: 1.6033x; 1.6033x over previous
import functools

import jax
import jax.numpy as jnp
from jax import lax
from jax.experimental import pallas as pl
from jax.experimental.pallas import tpu as pltpu
from jax.experimental.pallas import tpu_sc as plsc

NC = 2
NS = 16
L = 16
NW = NC * NS

BATCH = 16384
H_FEAT = 64
N_CLASSES = 2
BPW = BATCH // NW
NNODE = 1_000_000
SLABW = 256
NDC = 3906
TAIL0 = NDC * SLABW
NSLAB = 123
CAP = 1280
WIN = 128
DUMMY = 2 * BATCH
VROWS = 2 * BATCH + 8

_mesh = plsc.VectorSubcoreMesh(core_axis_name="c", subcore_axis_name="s")
_cparams = pltpu.CompilerParams(
    needs_layout_passes=False, use_tc_tiling_on_sc=True)


def _phase_a_body(embT_hbm, tail_hbm, ids_hbm, slots_hbm, out_hbm,
                  ids_v, slots_v, tail_v, slab_v, colbuf, slotw, sem):
    wid = lax.axis_index("s") * NC + lax.axis_index("c")
    pltpu.sync_copy(ids_hbm.at[wid], ids_v)
    pltpu.sync_copy(slots_hbm.at[wid], slots_v)
    pltpu.sync_copy(tail_hbm, tail_v)

    lane = lax.iota(jnp.int32, L)
    dummyvec = jnp.full((L,), DUMMY, jnp.int32)
    stc = (wid * NDC) // NW

    def initw(i, carry):
        slotw[0, pl.ds(i * L, L)] = dummyvec
        slotw[1, pl.ds(i * L, L)] = dummyvec
        return carry

    lax.fori_loop(0, WIN // L, initw, 0)

    def fire(c, slot):
        dc = stc + jnp.minimum(c, NSLAB - 1)
        pltpu.async_copy(embT_hbm.at[:, pl.ds(dc * SLABW, SLABW)],
                         slab_v.at[slot], sem)

    def drain(slot):
        pltpu.make_async_copy(embT_hbm.at[:, pl.ds(0, SLABW)],
                              slab_v.at[slot], sem).wait()

    def read_at(p, ref):
        pa = (p // L) * L
        vec = ref[pl.ds(pa, L)]
        sel = jnp.where(lane == (p % L), vec, 0)
        return jnp.sum(sel), vec

    def note_slot(pp):
        sval, _ = read_at(pp, slots_v)
        pr = pp % (2 * WIN)
        plsc.store_scatter(
            slotw,
            [jnp.full((L,), pr // WIN, jnp.int32),
             jnp.full((L,), pr % WIN, jnp.int32)],
            jnp.full((L,), sval, jnp.int32),
            mask=lane == 0)

    def flush(pfx):
        win = pfx % (2 * WIN)
        pltpu.sync_copy(
            colbuf.at[pl.ds(pl.multiple_of(win, WIN), WIN)],
            out_hbm.at[slotw.at[win // WIN]])

        def rst(i, carry):
            plsc.store_scatter(
                slotw,
                [jnp.full((L,), win // WIN, jnp.int32), i * L + lane],
                dummyvec)
            return carry

        lax.fori_loop(0, WIN // L, rst, 0)

    def slab_join(c, carry):
        p, pf = carry
        slot = c % 2
        fire(c + 1, (c + 1) % 2)
        drain(slot)
        slab_end = (stc + c + 1) * SLABW

        def cond(st):
            pp, _ = st
            idv, _ = read_at(pp, ids_v)
            return idv < slab_end

        def body(st):
            pp, _ = st
            idv, _ = read_at(pp, ids_v)
            l = idv - (stc + c) * SLABW
            row = pp % (2 * WIN)
            for k in range(H_FEAT // L):
                ch = plsc.load_gather(
                    slab_v, [jnp.full((L,), slot, jnp.int32),
                             k * L + lane,
                             jnp.full((L,), l, jnp.int32)])
                colbuf[row, pl.ds(k * L, L)] = ch
            note_slot(pp)
            return pp + 1, 0

        p, _ = lax.while_loop(cond, body, (p, 0))

        def fcond(st):
            pp, pfx = st
            return pfx + WIN <= pp

        def fbody(st):
            pp, pfx = st
            flush(pfx)
            return pp, pfx + WIN

        p, pf = lax.while_loop(fcond, fbody, (p, pf))
        return p, pf

    fire(0, 0)
    p, pf = lax.fori_loop(0, NSLAB, slab_join, (0, 0))
    drain(NSLAB % 2)

    def tcond(st):
        pp, _ = st
        idv, _ = read_at(pp, ids_v)
        return idv < (1 << 29)

    def tbody(st):
        pp, _ = st
        idv, _ = read_at(pp, ids_v)
        r = idv - TAIL0
        row = pp % (2 * WIN)
        for k in range(H_FEAT // L):
            ch = plsc.load_gather(
                tail_v, [jnp.full((L,), r, jnp.int32), k * L + lane])
            colbuf[row, pl.ds(k * L, L)] = ch
        note_slot(pp)
        return pp + 1, 0

    p, _ = lax.while_loop(tcond, tbody, (p, 0))

    @pl.when(pf < p)
    def _():
        flush(pf)

    @pl.when(pf + WIN < p)
    def _():
        flush(pf + WIN)


_phase_a = functools.partial(
    pl.kernel,
    out_type=jax.ShapeDtypeStruct((VROWS, 128), jnp.float32),
    mesh=_mesh,
    compiler_params=_cparams,
    scratch_types=[
        pltpu.VMEM((CAP,), jnp.int32),
        pltpu.VMEM((CAP,), jnp.int32),
        pltpu.VMEM((64, H_FEAT), jnp.float32),
        pltpu.VMEM((2, H_FEAT, SLABW), jnp.float32),
        pltpu.VMEM((2 * WIN, 128), jnp.float32),
        pltpu.VMEM((2, WIN), jnp.int32),
        pltpu.SemaphoreType.DMA,
    ],
)(_phase_a_body)


def _phase_b_body(vals_hbm, w_hbm, b_hbm, out_hbm,
                  chunk_v, wv, bv, outv, sem):
    wid = lax.axis_index("s") * NC + lax.axis_index("c")
    pltpu.sync_copy(w_hbm, wv)
    pltpu.sync_copy(b_hbm, bv)

    bvec = bv[...]
    nck = H_FEAT // L
    w0c = [wv[0, pl.ds(k * L, L)] for k in range(nck)]
    w1c = [wv[1, pl.ds(k * L, L)] for k in range(nck)]
    b0vec = jnp.full((L,), bvec[0], jnp.float32)
    b1vec = jnp.full((L,), bvec[1], jnp.float32)
    lane = lax.iota(jnp.int32, L)
    mask_hi = lane == (L - 1)

    NED = 128
    NCH = BPW // NED

    def fire(c, slot):
        cc = jnp.minimum(c, NCH - 1)
        pltpu.async_copy(
            vals_hbm.at[pl.ds(wid * 2 * BPW + cc * 2 * NED, 2 * NED)],
            chunk_v.at[slot], sem)

    def drain(slot):
        pltpu.make_async_copy(vals_hbm.at[pl.ds(0, 2 * NED)],
                              chunk_v.at[slot], sem).wait()

    def chunk(c, carry):
        slot = c % 2
        fire(c + 1, (c + 1) % 2)
        drain(slot)

        def grp(g, carry2):
            e0 = g * L
            for ln in range(L):
                r = 2 * (e0 + ln)
                t0 = None
                t1 = None
                for k in range(nck):
                    cd = (chunk_v[slot, r, pl.ds(k * L, L)]
                          * chunk_v[slot, r + 1, pl.ds(k * L, L)])
                    p0 = cd * w0c[k]
                    p1 = cd * w1c[k]
                    t0 = p0 if t0 is None else t0 + p0
                    t1 = p1 if t1 is None else t1 + p1
                cs0 = plsc.cumsum(t0) + b0vec
                cs1 = plsc.cumsum(t1) + b1vec
                oidx = jnp.full(
                    (L,), N_CLASSES * (c * NED + e0 + ln), jnp.int32)
                plsc.store_scatter(outv, [oidx], cs0, mask=mask_hi)
                plsc.store_scatter(outv, [oidx + 1], cs1, mask=mask_hi)
            return carry2

        lax.fori_loop(0, NED // L, grp, 0)
        return carry

    fire(0, 0)
    lax.fori_loop(0, NCH, chunk, 0)
    drain(NCH % 2)
    pltpu.sync_copy(outv, out_hbm.at[pl.ds(wid * BPW * N_CLASSES,
                                           BPW * N_CLASSES)])


_phase_b = functools.partial(
    pl.kernel,
    out_type=jax.ShapeDtypeStruct((BATCH * N_CLASSES,), jnp.float32),
    mesh=_mesh,
    compiler_params=_cparams,
    scratch_types=[
        pltpu.VMEM((2, 256, 128), jnp.float32),
        pltpu.VMEM((N_CLASSES, H_FEAT), jnp.float32),
        pltpu.VMEM((L,), jnp.float32),
        pltpu.VMEM((BATCH * N_CLASSES // NW,), jnp.float32),
        pltpu.SemaphoreType.DMA,
    ],
)(_phase_b_body)


def kernel(src_id, dst_id, embedding, W, b):
    src_id = src_id.astype(jnp.int32)
    dst_id = dst_id.astype(jnp.int32)
    ar = jnp.arange(BATCH, dtype=jnp.int32)
    ids_all = jnp.concatenate([src_id, dst_id])
    slots_all = jnp.concatenate([2 * ar, 2 * ar + 1])
    ids_s, slots_s = lax.sort([ids_all, slots_all], num_keys=1)

    bounds = ((jnp.arange(NW, dtype=jnp.int32) * NDC) // NW) * SLABW
    starts = jnp.searchsorted(
        ids_s, bounds, method="compare_all").astype(jnp.int32)

    rel = jnp.arange(CAP, dtype=jnp.int32)
    ends = jnp.concatenate(
        [starts[1:], jnp.array([2 * BATCH], jnp.int32)])
    gidx = starts[:, None] + rel[None, :]
    valid = gidx < ends[:, None]
    cl = jnp.clip(gidx, 0, 2 * BATCH - 1)
    ids_pad = jnp.where(valid, ids_s[cl], jnp.int32(1 << 29))
    slots_pad = jnp.where(valid, slots_s[cl], jnp.int32(DUMMY))

    tail_rows = lax.slice(embedding, (TAIL0, 0), (NNODE, H_FEAT))
    vals = _phase_a(embedding.T, tail_rows, ids_pad, slots_pad)
    b_pad = jnp.zeros((L,), jnp.float32).at[:N_CLASSES].set(b)
    out_flat = _phase_b(vals, W, b_pad)
    return out_flat.reshape(BATCH, N_CLASSES)


# speedup vs baseline: 1.8197x; 1.1349x over previous
import functools

import jax
import jax.numpy as jnp
from jax import lax
from jax.experimental import pallas as pl
from jax.experimental.pallas import tpu as pltpu
from jax.experimental.pallas import tpu_sc as plsc

NC = 2
NS = 16
L = 16
NW = NC * NS

BATCH = 16384
H_FEAT = 64
N_CLASSES = 2
BPW = BATCH // NW
NNODE = 1_000_000
SLABW = 512
NDC = 1953
TAIL0 = NDC * SLABW
NSLAB = 62
CAP = 1280
WIN = 128
DUMMY = 2 * BATCH
VROWS = 2 * BATCH + 8
SENTINEL = 1 << 29

_mesh = plsc.VectorSubcoreMesh(core_axis_name="c", subcore_axis_name="s")
_cparams = pltpu.CompilerParams(
    needs_layout_passes=False, use_tc_tiling_on_sc=True)


def _phase_a_body(embT_hbm, tail_hbm, ids_hbm, slots_hbm, starts_hbm,
                  out_hbm, ids_v, slots_v, tail_v, slab_v, colbuf, slotw,
                  starts_v, sem):
    wid = lax.axis_index("s") * NC + lax.axis_index("c")
    pltpu.sync_copy(starts_hbm, starts_v)
    pltpu.sync_copy(tail_hbm, tail_v)

    lane = lax.iota(jnp.int32, L)
    dummyvec = jnp.full((L,), DUMMY, jnp.int32)
    stc = (wid * NDC) // NW

    svec = jnp.where(lane == (wid % L),
                     starts_v[pl.ds((wid // L) * L, L)], 0)
    start = jnp.sum(svec)
    sa = pl.multiple_of((start // 8) * 8, 8)
    pltpu.sync_copy(ids_hbm.at[pl.ds(sa, CAP + 24)], ids_v)
    pltpu.sync_copy(slots_hbm.at[pl.ds(sa, CAP + 24)], slots_v)

    def initw(i, carry):
        slotw[0, pl.ds(i * L, L)] = dummyvec
        slotw[1, pl.ds(i * L, L)] = dummyvec
        return carry

    lax.fori_loop(0, WIN // L, initw, 0)

    def fire(c, slot):
        dc = jnp.minimum(stc + c, NDC - 1)
        pltpu.async_copy(embT_hbm.at[:, pl.ds(dc * SLABW, SLABW)],
                         slab_v.at[slot], sem)

    def drain(slot):
        pltpu.make_async_copy(embT_hbm.at[:, pl.ds(0, SLABW)],
                              slab_v.at[slot], sem).wait()

    def read_id(p):
        pa = (p // L) * L
        vec = ids_v[pl.ds(pa, L)]
        return jnp.sum(jnp.where(lane == (p % L), vec, 0))

    def note_slot(pp):
        pa = (pp // L) * L
        svals = slots_v[pl.ds(pa, L)]
        pr = pp % (2 * WIN)
        plsc.store_scatter(
            slotw,
            [jnp.full((L,), pr // WIN, jnp.int32),
             jnp.full((L,), pr % WIN, jnp.int32)],
            svals,
            mask=lane == (pp % L))

    def flush(pfx):
        win = pfx % (2 * WIN)
        pltpu.sync_copy(
            colbuf.at[pl.ds(pl.multiple_of(win, WIN), WIN)],
            out_hbm.at[slotw.at[win // WIN]])

        def rst(i, carry):
            plsc.store_scatter(
                slotw,
                [jnp.full((L,), win // WIN, jnp.int32), i * L + lane],
                dummyvec)
            return carry

        lax.fori_loop(0, WIN // L, rst, 0)

    def skcond(st):
        pp, idv = st
        return idv < stc * SLABW

    def skbody(st):
        pp, _ = st
        return pp + 1, read_id(pp + 1)

    p0, idv0 = lax.while_loop(skcond, skbody, (start - sa, read_id(start - sa)))
    pf0 = (p0 // WIN) * WIN

    def slab_join(c, carry):
        p, idv, pf = carry
        slot = c % 2
        fire(c + 1, (c + 1) % 2)
        drain(slot)
        slab_end = jnp.minimum(stc + c + 1, NDC) * SLABW

        def cond(st):
            pp, iv = st
            return iv < slab_end

        def body(st):
            pp, iv = st
            l = iv - (stc + c) * SLABW
            row = pp % (2 * WIN)
            for k in range(H_FEAT // L):
                ch = plsc.load_gather(
                    slab_v, [jnp.full((L,), slot, jnp.int32),
                             k * L + lane,
                             jnp.full((L,), l, jnp.int32)])
                colbuf[row, pl.ds(k * L, L)] = ch
            note_slot(pp)
            return pp + 1, read_id(pp + 1)

        p, idv = lax.while_loop(cond, body, (p, idv))

        def fcond(st):
            pp, pfx = st
            return pfx + WIN <= pp

        def fbody(st):
            pp, pfx = st
            flush(pfx)
            return pp, pfx + WIN

        p, pf = lax.while_loop(fcond, fbody, (p, pf))
        return p, idv, pf

    fire(0, 0)
    p, idv, pf = lax.fori_loop(0, NSLAB, slab_join, (p0, idv0, pf0))
    drain(NSLAB % 2)

    def tcond(st):
        pp, iv = st
        return jnp.logical_and(iv >= TAIL0, iv < SENTINEL)

    def tbody(st):
        pp, iv = st
        r = iv - TAIL0
        row = pp % (2 * WIN)
        for k in range(H_FEAT // L):
            ch = plsc.load_gather(
                tail_v, [jnp.full((L,), r, jnp.int32), k * L + lane])
            colbuf[row, pl.ds(k * L, L)] = ch
        note_slot(pp)
        return pp + 1, read_id(pp + 1)

    p, idv = lax.while_loop(tcond, tbody, (p, idv))

    @pl.when(pf < p)
    def _():
        flush(pf)

    @pl.when(pf + WIN < p)
    def _():
        flush(pf + WIN)


_phase_a = functools.partial(
    pl.kernel,
    out_type=jax.ShapeDtypeStruct((VROWS, 128), jnp.float32),
    mesh=_mesh,
    compiler_params=_cparams,
    scratch_types=[
        pltpu.VMEM((CAP + 24,), jnp.int32),
        pltpu.VMEM((CAP + 24,), jnp.int32),
        pltpu.VMEM((64, H_FEAT), jnp.float32),
        pltpu.VMEM((2, H_FEAT, SLABW), jnp.float32),
        pltpu.VMEM((2 * WIN, 128), jnp.float32),
        pltpu.VMEM((2, WIN), jnp.int32),
        pltpu.VMEM((NW,), jnp.int32),
        pltpu.SemaphoreType.DMA,
    ],
)(_phase_a_body)


def _phase_b_body(vals_hbm, w_hbm, b_hbm, out_hbm,
                  chunk_v, wv, bv, outv, sem):
    wid = lax.axis_index("s") * NC + lax.axis_index("c")
    pltpu.sync_copy(w_hbm, wv)
    pltpu.sync_copy(b_hbm, bv)

    bvec = bv[...]
    nck = H_FEAT // L
    w0c = [wv[0, pl.ds(k * L, L)] for k in range(nck)]
    w1c = [wv[1, pl.ds(k * L, L)] for k in range(nck)]
    b0vec = jnp.full((L,), bvec[0], jnp.float32)
    b1vec = jnp.full((L,), bvec[1], jnp.float32)
    lane = lax.iota(jnp.int32, L)
    mask_hi = lane == (L - 1)

    NED = 128
    NCH = BPW // NED

    def fire(c, slot):
        cc = jnp.minimum(c, NCH - 1)
        pltpu.async_copy(
            vals_hbm.at[pl.ds(wid * 2 * BPW + cc * 2 * NED, 2 * NED)],
            chunk_v.at[slot], sem)

    def drain(slot):
        pltpu.make_async_copy(vals_hbm.at[pl.ds(0, 2 * NED)],
                              chunk_v.at[slot], sem).wait()

    def chunk(c, carry):
        slot = c % 2
        fire(c + 1, (c + 1) % 2)
        drain(slot)

        def grp(g, carry2):
            e0 = g * L
            for ln in range(L):
                r = 2 * (e0 + ln)
                t0 = None
                t1 = None
                for k in range(nck):
                    cd = (chunk_v[slot, r, pl.ds(k * L, L)]
                          * chunk_v[slot, r + 1, pl.ds(k * L, L)])
                    p0 = cd * w0c[k]
                    p1 = cd * w1c[k]
                    t0 = p0 if t0 is None else t0 + p0
                    t1 = p1 if t1 is None else t1 + p1
                cs0 = plsc.cumsum(t0) + b0vec
                cs1 = plsc.cumsum(t1) + b1vec
                oidx = jnp.full(
                    (L,), N_CLASSES * (c * NED + e0 + ln), jnp.int32)
                plsc.store_scatter(outv, [oidx], cs0, mask=mask_hi)
                plsc.store_scatter(outv, [oidx + 1], cs1, mask=mask_hi)
            return carry2

        lax.fori_loop(0, NED // L, grp, 0)
        return carry

    fire(0, 0)
    lax.fori_loop(0, NCH, chunk, 0)
    drain(NCH % 2)
    pltpu.sync_copy(outv, out_hbm.at[pl.ds(wid * BPW * N_CLASSES,
                                           BPW * N_CLASSES)])


_phase_b = functools.partial(
    pl.kernel,
    out_type=jax.ShapeDtypeStruct((BATCH * N_CLASSES,), jnp.float32),
    mesh=_mesh,
    compiler_params=_cparams,
    scratch_types=[
        pltpu.VMEM((2, 256, 128), jnp.float32),
        pltpu.VMEM((N_CLASSES, H_FEAT), jnp.float32),
        pltpu.VMEM((L,), jnp.float32),
        pltpu.VMEM((BATCH * N_CLASSES // NW,), jnp.float32),
        pltpu.SemaphoreType.DMA,
    ],
)(_phase_b_body)


def kernel(src_id, dst_id, embedding, W, b):
    src_id = src_id.astype(jnp.int32)
    dst_id = dst_id.astype(jnp.int32)
    ar = jnp.arange(BATCH, dtype=jnp.int32)
    ids_all = jnp.concatenate([src_id, dst_id])
    slots_all = jnp.concatenate([2 * ar, 2 * ar + 1])
    ids_s, slots_s = lax.sort([ids_all, slots_all], num_keys=1)

    ids_ext = jnp.concatenate(
        [ids_s, jnp.full((CAP + 24,), SENTINEL, jnp.int32)])
    slots_ext = jnp.concatenate(
        [slots_s, jnp.full((CAP + 24,), DUMMY, jnp.int32)])

    bounds = ((jnp.arange(NW, dtype=jnp.int32) * NDC) // NW) * SLABW
    starts = jnp.searchsorted(
        ids_s, bounds, method="compare_all").astype(jnp.int32)

    tail_rows = lax.slice(embedding, (TAIL0, 0), (NNODE, H_FEAT))
    vals = _phase_a(embedding.T, tail_rows, ids_ext, slots_ext, starts)
    b_pad = jnp.zeros((L,), jnp.float32).at[:N_CLASSES].set(b)
    out_flat = _phase_b(vals, W, b_pad)
    return out_flat.reshape(BATCH, N_CLASSES)
